# baseline (speedup 1.0000x reference)
import functools

import jax
import jax.numpy as jnp
from jax import lax
from jax.experimental import pallas as pl
from jax.experimental.pallas import tpu as pltpu
from jax.experimental.pallas import tpu_sc as plsc

_GRAY = 127.5
_N_IMG = 16
_H = 512
_W = 512
_VALID = _H - 2
_POS_PER_HIST = _N_IMG * _VALID * _VALID

_N_TILES = 32
_WORDS = _N_IMG * _H * _W
_WORDS_PER_TILE = _WORDS // _N_TILES
_ROWS = 64
_CHUNK = _ROWS * _W
_N_CHUNK = _WORDS_PER_TILE // _CHUNK
_HALF_OFF = 528 * 16
_HIST_WORDS = 2 * _HALF_OFF
_CNT_HALF = 640
_CNT_ROW = 2 * _CNT_HALF


def _codes_body(inp_ref, tgt_ref, cin_ref, ctgt_ref):
    for src, dst in ((inp_ref, cin_ref), (tgt_ref, ctgt_ref)):
        x = src[0]
        xb = (((x * 0.5 + 0.5) * 255.0) > _GRAY).astype(jnp.int32)
        rc = (xb[:, 0:510] << 2) + (xb[:, 1:511] << 1) + xb[:, 2:512]
        code = (rc[0:510] << 6) + (rc[1:511] << 3) + rc[2:512]
        code = jnp.concatenate(
            [code, jnp.full((_VALID, 2), 512, jnp.int32)], axis=1)
        code = jnp.concatenate(
            [code, jnp.full((2, _W), 512, jnp.int32)], axis=0)
        dst[0] = code


def _codes(inp, tgt, i0, n):
    def imap(i, _i0=i0):
        return (i + _i0, 0, 0)

    return pl.pallas_call(
        _codes_body,
        grid=(n,),
        in_specs=[
            pl.BlockSpec((1, _H, _W), imap),
            pl.BlockSpec((1, _H, _W), imap),
        ],
        out_specs=[
            pl.BlockSpec((1, _H, _W), lambda i: (i, 0, 0)),
            pl.BlockSpec((1, _H, _W), lambda i: (i, 0, 0)),
        ],
        out_shape=[
            jax.ShapeDtypeStruct((n, _H, _W), jnp.int32),
            jax.ShapeDtypeStruct((n, _H, _W), jnp.int32),
        ],
    )(inp, tgt)


def _hist_body(cin, ctgt, out_hbm, buf, hist, counts, sem0, sem1,
               n_img=_N_IMG):
    wid = lax.axis_index("s") * 2 + lax.axis_index("c")
    lane = lax.iota(jnp.int32, 16)
    ones = jnp.ones((16,), jnp.float32)

    @plsc.parallel_loop(0, _HIST_WORDS // 16, unroll=8)
    def _zero(i):
        hist[pl.ds(i * 16, 16)] = jnp.zeros((16,), jnp.float32)

    tiles_per_img = _N_TILES // n_img
    rows_per_tile = _H // tiles_per_img
    n_chunk = rows_per_tile // _ROWS
    img = wid // tiles_per_img
    r0 = (wid % tiles_per_img) * rows_per_tile
    chunks = [(half, src, k)
              for half, src in ((0, cin), (1, ctgt))
              for k in range(n_chunk)]
    sems = (sem0, sem1)
    n = len(chunks)
    _, src0, k0 = chunks[0]
    pending = pltpu.async_copy(
        src0.at[img, pl.ds(r0 + k0 * _ROWS, _ROWS), :], buf.at[0], sems[0])
    for ci in range(n):
        half, _, _ = chunks[ci]
        s = ci % 2
        if ci + 1 < n:
            _, nsrc, nk = chunks[ci + 1]
            nxt = pltpu.async_copy(
                nsrc.at[img, pl.ds(r0 + nk * _ROWS, _ROWS), :],
                buf.at[1 - s], sems[1 - s])
        pending.wait()

        @plsc.parallel_loop(0, _CHUNK // 16, unroll=8)
        def _chunk(j, _off=half * _HALF_OFF, _s=s):
            r = j >> 5
            c = (j & 31) << 4
            codes = buf[_s, r, pl.ds(c, 16)]
            idx = (codes << 4) + lane + _off
            plsc.addupdate_scatter(hist, [idx], ones)

        if ci + 1 < n:
            pending = nxt

    for half in range(2):
        hoff = half * _HALF_OFF
        coff = half * _CNT_HALF

        def red_body(g, _):
            addr0 = hoff + ((g * 16 + lane) << 4)
            acc = jnp.zeros((16,), jnp.float32)
            for l in range(16):
                acc = acc + plsc.load_gather(hist, [addr0 + l])
            counts[pl.ds(coff + g * 16, 16)] = acc
            return 0

        lax.fori_loop(0, 33, red_body, 0)

    pltpu.sync_copy(counts, out_hbm.at[wid])


@functools.cache
def _hist(n_img):
    return pl.kernel(
        functools.partial(_hist_body, n_img=n_img),
        out_type=jax.ShapeDtypeStruct((_N_TILES, _CNT_ROW), jnp.float32),
        mesh=plsc.VectorSubcoreMesh(core_axis_name="c", subcore_axis_name="s"),
        compiler_params=pltpu.CompilerParams(needs_layout_passes=False),
        scratch_types=[
            pltpu.VMEM((2, _ROWS, _W), jnp.int32),
            pltpu.VMEM((_HIST_WORDS,), jnp.float32),
            pltpu.VMEM((_CNT_ROW,), jnp.float32),
            pltpu.SemaphoreType.DMA,
            pltpu.SemaphoreType.DMA,
        ],
    )

_MSE_SCALE = 1.0 / (float(_POS_PER_HIST) ** 2 * 512.0 * float(_N_IMG))


def _mse_body(pa_ref, pb_ref, out_ref):
    s = (jnp.sum(pa_ref[...], axis=0, keepdims=True)
         + jnp.sum(pb_ref[...], axis=0, keepdims=True))
    d = s[:, 0:512] - s[:, _CNT_HALF:_CNT_HALF + 512]
    out_ref[0, 0] = jnp.sum(d * d) * _MSE_SCALE


def _mse(pa, pb):
    return pl.pallas_call(
        _mse_body,
        out_specs=pl.BlockSpec(memory_space=pltpu.SMEM),
        out_shape=jax.ShapeDtypeStruct((1, 1), jnp.float32),
    )(pa, pb)


def kernel(input, target):
    inp = input.reshape(_N_IMG, _H, _W)
    tgt = target.reshape(_N_IMG, _H, _W)
    nb = _N_IMG // 2
    cin_a, ctgt_a = _codes(inp, tgt, 0, nb)
    parts_a = _hist(nb)(cin_a, ctgt_a)
    cin_b, ctgt_b = _codes(inp, tgt, nb, nb)
    parts_b = _hist(nb)(cin_b, ctgt_b)
    return _mse(parts_a, parts_b)[0, 0]


# speedup vs baseline: 1.1172x; 1.1172x over previous
import functools

import jax
import jax.numpy as jnp
from jax import lax
from jax.experimental import pallas as pl
from jax.experimental.pallas import tpu as pltpu
from jax.experimental.pallas import tpu_sc as plsc

_BIN_THRESH = float(2.0 ** -24)
_N_IMG = 16
_H = 512
_W = 512
_VALID = _H - 2
_POS_PER_HIST = _N_IMG * _VALID * _VALID

_N_TILES = 32
_WORDS = _N_IMG * (_H // 2) * _W
_WORDS_PER_TILE = _WORDS // _N_TILES
_ROWS = 64
_CHUNK = _ROWS * _W
_N_CHUNK = _WORDS_PER_TILE // _CHUNK
_HALF_OFF = 528 * 16
_HIST_WORDS = 2 * _HALF_OFF
_CNT_HALF = 640
_CNT_ROW = 2 * _CNT_HALF


def _codes_body(inp_ref, tgt_ref, cin_ref, ctgt_ref):
    for src, dst in ((inp_ref, cin_ref), (tgt_ref, ctgt_ref)):
        x = src[0]
        xb = (x > _BIN_THRESH).astype(jnp.int32)
        rc = (xb[:, 0:510] << 2) + (xb[:, 1:511] << 1) + xb[:, 2:512]
        code = (rc[0:510] << 6) + (rc[1:511] << 3) + rc[2:512]
        code = jnp.concatenate(
            [code, jnp.full((_VALID, 2), 512, jnp.int32)], axis=1)
        code = jnp.concatenate(
            [code, jnp.full((2, _W), 512, jnp.int32)], axis=0)
        dst[0] = code[0:256] | (code[256:512] << 16)


def _codes(inp, tgt):
    return pl.pallas_call(
        _codes_body,
        grid=(_N_IMG,),
        in_specs=[
            pl.BlockSpec((1, _H, _W), lambda i: (i, 0, 0)),
            pl.BlockSpec((1, _H, _W), lambda i: (i, 0, 0)),
        ],
        out_specs=[
            pl.BlockSpec((1, _H // 2, _W), lambda i: (i, 0, 0)),
            pl.BlockSpec((1, _H // 2, _W), lambda i: (i, 0, 0)),
        ],
        out_shape=[
            jax.ShapeDtypeStruct((_N_IMG, _H // 2, _W), jnp.int32),
            jax.ShapeDtypeStruct((_N_IMG, _H // 2, _W), jnp.int32),
        ],
    )(inp, tgt)


def _hist_body(cin, ctgt, out_hbm, buf, hist, counts, sem0, sem1):
    wid = lax.axis_index("s") * 2 + lax.axis_index("c")
    lane = lax.iota(jnp.int32, 16)
    ones = jnp.ones((16,), jnp.float32)

    @plsc.parallel_loop(0, _HIST_WORDS // 16, unroll=8)
    def _zero(i):
        hist[pl.ds(i * 16, 16)] = jnp.zeros((16,), jnp.float32)

    img = wid >> 1
    r0 = (wid & 1) * 128
    chunks = [(half, src, k)
              for half, src in ((0, cin), (1, ctgt))
              for k in range(_N_CHUNK)]
    sems = (sem0, sem1)
    n = len(chunks)
    _, src0, k0 = chunks[0]
    pending = pltpu.async_copy(
        src0.at[img, pl.ds(r0 + k0 * _ROWS, _ROWS), :], buf.at[0], sems[0])
    for ci in range(n):
        half, _, _ = chunks[ci]
        s = ci % 2
        if ci + 1 < n:
            _, nsrc, nk = chunks[ci + 1]
            nxt = pltpu.async_copy(
                nsrc.at[img, pl.ds(r0 + nk * _ROWS, _ROWS), :],
                buf.at[1 - s], sems[1 - s])
        pending.wait()

        @plsc.parallel_loop(0, _CHUNK // 16, unroll=8)
        def _chunk(j, _off=half * _HALF_OFF, _s=s):
            r = j >> 5
            c = (j & 31) << 4
            w = buf[_s, r, pl.ds(c, 16)]
            lane_off = lane + _off
            idx_lo = ((w << 4) & 0xFFFF0) + lane_off
            plsc.addupdate_scatter(hist, [idx_lo], ones)
            idx_hi = (lax.shift_right_logical(w, 12) & 0xFFFF0) + lane_off
            plsc.addupdate_scatter(hist, [idx_hi], ones)

        if ci + 1 < n:
            pending = nxt

    for half in range(2):
        hoff = half * _HALF_OFF
        coff = half * _CNT_HALF

        def red_body(g, _):
            addr0 = hoff + ((g * 16 + lane) << 4)
            acc = jnp.zeros((16,), jnp.float32)
            for l in range(16):
                acc = acc + plsc.load_gather(hist, [addr0 + l])
            counts[pl.ds(coff + g * 16, 16)] = acc
            return 0

        lax.fori_loop(0, 33, red_body, 0)

    pltpu.sync_copy(counts, out_hbm.at[wid])


@functools.cache
def _hist():
    return pl.kernel(
        _hist_body,
        out_type=jax.ShapeDtypeStruct((_N_TILES, _CNT_ROW), jnp.float32),
        mesh=plsc.VectorSubcoreMesh(core_axis_name="c", subcore_axis_name="s"),
        compiler_params=pltpu.CompilerParams(needs_layout_passes=False),
        scratch_types=[
            pltpu.VMEM((2, _ROWS, _W), jnp.int32),
            pltpu.VMEM((_HIST_WORDS,), jnp.float32),
            pltpu.VMEM((_CNT_ROW,), jnp.float32),
            pltpu.SemaphoreType.DMA,
            pltpu.SemaphoreType.DMA,
        ],
    )

_MSE_SCALE = 1.0 / (float(_POS_PER_HIST) ** 2 * 512.0 * float(_N_IMG))


def _mse_body(p_ref, out_ref):
    s = jnp.sum(p_ref[...], axis=0, keepdims=True)
    d = s[:, 0:512] - s[:, _CNT_HALF:_CNT_HALF + 512]
    out_ref[0, 0] = jnp.sum(d * d) * _MSE_SCALE


def _mse(parts):
    return pl.pallas_call(
        _mse_body,
        out_specs=pl.BlockSpec(memory_space=pltpu.SMEM),
        out_shape=jax.ShapeDtypeStruct((1, 1), jnp.float32),
    )(parts)


def kernel(input, target):
    inp = input.reshape(_N_IMG, _H, _W)
    tgt = target.reshape(_N_IMG, _H, _W)
    cin, ctgt = _codes(inp, tgt)
    parts = _hist()(cin, ctgt)
    return _mse(parts)[0, 0]


# speedup vs baseline: 1.1192x; 1.0018x over previous
import functools

import jax
import jax.numpy as jnp
from jax import lax
from jax.experimental import pallas as pl
from jax.experimental.pallas import tpu as pltpu
from jax.experimental.pallas import tpu_sc as plsc

_BIN_THRESH = float(2.0 ** -24)
_N_IMG = 16
_H = 512
_W = 512
_VALID = _H - 2
_POS_PER_HIST = _N_IMG * _VALID * _VALID

_N_TILES = 32
_WORDS = _N_IMG * (_H // 2) * _W
_WORDS_PER_TILE = _WORDS // _N_TILES
_ROWS = 32
_CHUNK = _ROWS * _W
_N_CHUNK = _WORDS_PER_TILE // _CHUNK
_HALF_OFF = 528 * 16
_HIST_WORDS = 2 * _HALF_OFF
_CNT_HALF = 640
_CNT_ROW = 2 * _CNT_HALF


def _codes_body(inp_ref, tgt_ref, cin_ref, ctgt_ref):
    for src, dst in ((inp_ref, cin_ref), (tgt_ref, ctgt_ref)):
        x = src[0]
        xb = (x > _BIN_THRESH).astype(jnp.int32)
        rc = (xb[:, 0:510] << 2) + (xb[:, 1:511] << 1) + xb[:, 2:512]
        code = (rc[0:510] << 6) + (rc[1:511] << 3) + rc[2:512]
        code = jnp.concatenate(
            [code, jnp.full((_VALID, 2), 512, jnp.int32)], axis=1)
        code = jnp.concatenate(
            [code, jnp.full((2, _W), 512, jnp.int32)], axis=0)
        dst[0] = code[0:256] | (code[256:512] << 16)


def _codes(inp, tgt):
    return pl.pallas_call(
        _codes_body,
        grid=(_N_IMG,),
        in_specs=[
            pl.BlockSpec((1, _H, _W), lambda i: (i, 0, 0)),
            pl.BlockSpec((1, _H, _W), lambda i: (i, 0, 0)),
        ],
        out_specs=[
            pl.BlockSpec((1, _H // 2, _W), lambda i: (i, 0, 0)),
            pl.BlockSpec((1, _H // 2, _W), lambda i: (i, 0, 0)),
        ],
        out_shape=[
            jax.ShapeDtypeStruct((_N_IMG, _H // 2, _W), jnp.int32),
            jax.ShapeDtypeStruct((_N_IMG, _H // 2, _W), jnp.int32),
        ],
    )(inp, tgt)


def _hist_body(cin, ctgt, out_hbm, buf, hist, counts, sem0, sem1):
    wid = lax.axis_index("s") * 2 + lax.axis_index("c")
    lane = lax.iota(jnp.int32, 16)
    ones = jnp.ones((16,), jnp.float32)

    @plsc.parallel_loop(0, _HIST_WORDS // 16, unroll=8)
    def _zero(i):
        hist[pl.ds(i * 16, 16)] = jnp.zeros((16,), jnp.float32)

    img = wid >> 1
    r0 = (wid & 1) * 128
    chunks = [(half, src, k)
              for half, src in ((0, cin), (1, ctgt))
              for k in range(_N_CHUNK)]
    sems = (sem0, sem1)
    n = len(chunks)
    _, src0, k0 = chunks[0]
    pending = pltpu.async_copy(
        src0.at[img, pl.ds(r0 + k0 * _ROWS, _ROWS), :], buf.at[0], sems[0])
    for ci in range(n):
        half, _, _ = chunks[ci]
        s = ci % 2
        if ci + 1 < n:
            _, nsrc, nk = chunks[ci + 1]
            nxt = pltpu.async_copy(
                nsrc.at[img, pl.ds(r0 + nk * _ROWS, _ROWS), :],
                buf.at[1 - s], sems[1 - s])
        pending.wait()

        @plsc.parallel_loop(0, _CHUNK // 16, unroll=16)
        def _chunk(j, _off=half * _HALF_OFF, _s=s):
            r = j >> 5
            c = (j & 31) << 4
            w = buf[_s, r, pl.ds(c, 16)]
            lane_off = lane + _off
            idx_lo = ((w << 4) & 0xFFFF0) + lane_off
            plsc.addupdate_scatter(hist, [idx_lo], ones)
            idx_hi = (lax.shift_right_logical(w, 12) & 0xFFFF0) + lane_off
            plsc.addupdate_scatter(hist, [idx_hi], ones)

        if ci + 1 < n:
            pending = nxt

    for half in range(2):
        hoff = half * _HALF_OFF
        coff = half * _CNT_HALF

        def red_body(g, _):
            addr0 = hoff + ((g * 16 + lane) << 4)
            acc = jnp.zeros((16,), jnp.float32)
            for l in range(16):
                acc = acc + plsc.load_gather(hist, [addr0 + l])
            counts[pl.ds(coff + g * 16, 16)] = acc
            return 0

        lax.fori_loop(0, 33, red_body, 0)

    pltpu.sync_copy(counts, out_hbm.at[wid])


@functools.cache
def _hist():
    return pl.kernel(
        _hist_body,
        out_type=jax.ShapeDtypeStruct((_N_TILES, _CNT_ROW), jnp.float32),
        mesh=plsc.VectorSubcoreMesh(core_axis_name="c", subcore_axis_name="s"),
        compiler_params=pltpu.CompilerParams(needs_layout_passes=False),
        scratch_types=[
            pltpu.VMEM((2, _ROWS, _W), jnp.int32),
            pltpu.VMEM((_HIST_WORDS,), jnp.float32),
            pltpu.VMEM((_CNT_ROW,), jnp.float32),
            pltpu.SemaphoreType.DMA,
            pltpu.SemaphoreType.DMA,
        ],
    )

_MSE_SCALE = 1.0 / (float(_POS_PER_HIST) ** 2 * 512.0 * float(_N_IMG))


def _mse_body(p_ref, out_ref):
    s = jnp.sum(p_ref[...], axis=0, keepdims=True)
    d = s[:, 0:512] - s[:, _CNT_HALF:_CNT_HALF + 512]
    out_ref[0, 0] = jnp.sum(d * d) * _MSE_SCALE


def _mse(parts):
    return pl.pallas_call(
        _mse_body,
        out_specs=pl.BlockSpec(memory_space=pltpu.SMEM),
        out_shape=jax.ShapeDtypeStruct((1, 1), jnp.float32),
    )(parts)


def kernel(input, target):
    inp = input.reshape(_N_IMG, _H, _W)
    tgt = target.reshape(_N_IMG, _H, _W)
    cin, ctgt = _codes(inp, tgt)
    parts = _hist()(cin, ctgt)
    return _mse(parts)[0, 0]
